# baseline (speedup 1.0000x reference)
import functools

import jax
import jax.numpy as jnp
from jax import lax
from jax.experimental import pallas as pl
from jax.experimental.pallas import tpu as pltpu
from jax.experimental.pallas import tpu_sc as plsc

_J = 8
_M = 8
_JR = 28
_P = 56
_N = 8
_COLS = 4096
_CHUNK = 1024
_NCHUNK = _COLS // _CHUNK
_LANES = 16
_CV = _CHUNK // _LANES


def _sc_kernel(x_hbm, l_hbm, m1_hbm, ir_hbm, m2_hbm, pa_hbm, out_hbm,
               xs, zs, lv, m1v, irv, m2v, pav, sem):
    wid = lax.axis_index("s") * 2 + lax.axis_index("c")
    n = wid // _NCHUNK
    chunk = wid % _NCHUNK
    col0 = chunk * _CHUNK

    pltpu.sync_copy(l_hbm, lv)
    pltpu.sync_copy(m1_hbm, m1v)
    pltpu.sync_copy(ir_hbm, irv)
    pltpu.sync_copy(m2_hbm, m2v)
    pltpu.sync_copy(pa_hbm, pav)
    pltpu.sync_copy(x_hbm.at[n, :, pl.ds(col0, _CHUNK)], xs)

    lanes = lax.iota(jnp.int32, _LANES)
    zero = jnp.zeros((_LANES,), jnp.float32)

    def _zero_row(r, carry):
        rvec = jnp.zeros((_LANES,), jnp.int32) + r
        for c in range(_CV):
            plsc.store_scatter(zs, [rvec, c * _LANES + lanes], zero)
        return carry
    lax.fori_loop(0, _M * _J, _zero_row, 0)

    def _accum(k, carry):
        kvec = jnp.zeros((_LANES,), jnp.int32) + k
        m1s = plsc.load_gather(m1v, [kvec])
        irs = plsc.load_gather(irv, [kvec])
        for cb in range(4):
            xregs = [
                plsc.load_gather(
                    xs, [kvec, cb * 256 + c * _LANES + lanes])
                for c in range(16)
            ]
            for a in range(_J):
                coeff = plsc.load_gather(lv, [a * _J + irs])
                rowvec = m1s * _J + a
                for c in range(16):
                    plsc.addupdate_scatter(
                        zs, [rowvec, cb * 256 + c * _LANES + lanes],
                        coeff * xregs[c])
        return carry
    lax.fori_loop(0, _JR, _accum, 0)

    copies = []
    for pb in range(4):
        base = pb * _LANES
        cnt = min(_LANES, _P - base)
        if cnt <= 0:
            break
        m2vec = m2v[pl.ds(base, _LANES)]
        pavec = pav[pl.ds(base, _LANES)]
        rowvec = m2vec * _J + pavec
        for j in range(cnt):
            row = jnp.max(jnp.where(lanes == j, rowvec, -1))
            copies.append(pltpu.async_copy(
                zs.at[row],
                out_hbm.at[n, base + j, pl.ds(col0, _CHUNK)], sem))
    for cp in copies:
        cp.wait()


def kernel(x, L, mapping1, idx_r, mapping2, sums, p_next_a, p_next_b):
    del sums, p_next_b
    B, n, Jr, T_, two = x.shape
    nt = T_ // 128
    x4 = x.reshape(n, Jr, nt, 128, two).transpose(0, 1, 2, 4, 3).reshape(
        n, Jr, _COLS)
    lflat = L.reshape(_J * _J)
    m1 = jnp.pad(mapping1.astype(jnp.int32), (0, 4))
    ir = jnp.pad(idx_r.astype(jnp.int32), (0, 4))
    m2 = jnp.pad(mapping2.astype(jnp.int32), (0, 8))
    pa = jnp.pad(p_next_a.astype(jnp.int32), (0, 8))

    mesh = plsc.VectorSubcoreMesh(core_axis_name="c", subcore_axis_name="s")
    run = functools.partial(
        pl.kernel,
        mesh=mesh,
        compiler_params=pltpu.CompilerParams(needs_layout_passes=False),
        out_type=jax.ShapeDtypeStruct((n, _P, _COLS), jnp.float32),
        scratch_types=[
            pltpu.VMEM((_JR, _CHUNK), jnp.float32),
            pltpu.VMEM((_M * _J, _CHUNK), jnp.float32),
            pltpu.VMEM((_J * _J,), jnp.float32),
            pltpu.VMEM((32,), jnp.int32),
            pltpu.VMEM((32,), jnp.int32),
            pltpu.VMEM((64,), jnp.int32),
            pltpu.VMEM((64,), jnp.int32),
            pltpu.SemaphoreType.DMA,
        ],
    )(_sc_kernel)
    out4 = run(x4, lflat, m1, ir, m2, pa)
    out = out4.reshape(n, _P, nt, two, 128).transpose(0, 1, 2, 4, 3).reshape(
        B, n, _P, T_, two)
    return out


# speedup vs baseline: 1.0874x; 1.0874x over previous
import functools

import jax
import jax.numpy as jnp
from jax import lax
from jax.experimental import pallas as pl
from jax.experimental.pallas import tpu as pltpu
from jax.experimental.pallas import tpu_sc as plsc

_J = 8
_M = 8
_JR = 28
_P = 56
_N = 8
_COLS = 4096
_CHUNK = 1024
_NCHUNK = _COLS // _CHUNK
_LANES = 16
_CV = _CHUNK // _LANES
_OFF_L = 0
_OFF_M1 = 64
_OFF_IR = 96
_OFF_M2 = 128
_OFF_PA = 192


def _sc_kernel(x_hbm, tbl_hbm, out_hbm, xs, zs, tbl, sem, xsem):
    wid = lax.axis_index("s") * 2 + lax.axis_index("c")
    n = wid // _NCHUNK
    chunk = wid % _NCHUNK
    col0 = chunk * _CHUNK

    xcopy = pltpu.async_copy(x_hbm.at[n, :, pl.ds(col0, _CHUNK)], xs, xsem)
    pltpu.sync_copy(tbl_hbm, tbl)

    lanes = lax.iota(jnp.int32, _LANES)
    zero = jnp.zeros((_LANES,), jnp.float32)

    def _zero_row(r):
        rvec = jnp.zeros((_LANES,), jnp.int32) + r
        for c in range(_CV):
            plsc.store_scatter(zs, [rvec, c * _LANES + lanes], zero)
    plsc.parallel_loop(0, _M * _J, 1, unroll=2)(_zero_row)

    xcopy.wait()

    def _accum(k, carry):
        kvec = jnp.zeros((_LANES,), jnp.int32) + k
        m1s = plsc.load_gather(tbl, [_OFF_M1 + kvec])
        irs = plsc.load_gather(tbl, [_OFF_IR + kvec])
        for cb in range(4):
            xregs = [
                plsc.load_gather(
                    xs, [kvec, cb * 256 + c * _LANES + lanes])
                for c in range(16)
            ]
            for a in range(_J):
                coeff = plsc.bitcast(
                    plsc.load_gather(tbl, [_OFF_L + a * _J + irs]),
                    jnp.float32)
                rowvec = m1s * _J + a
                for c in range(16):
                    plsc.addupdate_scatter(
                        zs, [rowvec, cb * 256 + c * _LANES + lanes],
                        coeff * xregs[c])
        return carry
    lax.fori_loop(0, _JR, _accum, 0)

    copies = []
    for pb in range(4):
        base = pb * _LANES
        cnt = min(_LANES, _P - base)
        if cnt <= 0:
            break
        m2vec = plsc.load_gather(tbl, [_OFF_M2 + base + lanes])
        pavec = plsc.load_gather(tbl, [_OFF_PA + base + lanes])
        rowvec = m2vec * _J + pavec
        for j in range(cnt):
            row = jnp.max(jnp.where(lanes == j, rowvec, -1))
            copies.append(pltpu.async_copy(
                zs.at[row],
                out_hbm.at[n, base + j, pl.ds(col0, _CHUNK)], sem))
    for cp in copies:
        cp.wait()


def kernel(x, L, mapping1, idx_r, mapping2, sums, p_next_a, p_next_b):
    del sums, p_next_b
    B, n, Jr, T_, two = x.shape
    nt = T_ // 128
    x4 = x.reshape(n, Jr, nt, 128, two).transpose(0, 1, 2, 4, 3).reshape(
        n, Jr, _COLS)
    tbl = jnp.concatenate([
        jax.lax.bitcast_convert_type(L.reshape(_J * _J), jnp.int32),
        jnp.pad(mapping1.astype(jnp.int32), (0, 4)),
        jnp.pad(idx_r.astype(jnp.int32), (0, 4)),
        jnp.pad(mapping2.astype(jnp.int32), (0, 8)),
        jnp.pad(p_next_a.astype(jnp.int32), (0, 8)),
    ])

    mesh = plsc.VectorSubcoreMesh(core_axis_name="c", subcore_axis_name="s")
    run = functools.partial(
        pl.kernel,
        mesh=mesh,
        compiler_params=pltpu.CompilerParams(needs_layout_passes=False),
        out_type=jax.ShapeDtypeStruct((n, _P, _COLS), jnp.float32),
        scratch_types=[
            pltpu.VMEM((_JR, _CHUNK), jnp.float32),
            pltpu.VMEM((_M * _J, _CHUNK), jnp.float32),
            pltpu.VMEM((256,), jnp.int32),
            pltpu.SemaphoreType.DMA,
            pltpu.SemaphoreType.DMA,
        ],
    )(_sc_kernel)
    out4 = run(x4, tbl)
    out = out4.reshape(n, _P, nt, two, 128).transpose(0, 1, 2, 4, 3).reshape(
        B, n, _P, T_, two)
    return out


# speedup vs baseline: 8.7020x; 8.0026x over previous
import jax
import jax.numpy as jnp
from jax.experimental import pallas as pl
from jax.experimental.pallas import tpu as pltpu

_J = 8
_M = 8
_JR = 28
_P = 56
_N = 8
_T = 2048


def _mix_kernel(x_ref, L_ref, m1_ref, ir_ref, m2_ref, pa_ref, out_ref, w_ref):
    @pl.when(pl.program_id(0) == 0)
    def _build_w():
        m1 = m1_ref[0:1, :]
        ir = ir_ref[0:1, :]
        m2 = m2_ref[:, 0:1]
        pa = pa_ref[:, 0:1]
        mask = (m2 == m1).astype(jnp.float32)
        oh_a = (pa == jax.lax.broadcasted_iota(jnp.int32, (_P, _J), 1)).astype(
            jnp.float32)
        oh_r = (jax.lax.broadcasted_iota(jnp.int32, (_J, _JR), 0) == ir).astype(
            jnp.float32)
        lg = jnp.dot(jnp.dot(oh_a, L_ref[:, :],
                             preferred_element_type=jnp.float32),
                     oh_r, preferred_element_type=jnp.float32)
        w_ref[:, :] = lg * mask

    w = w_ref[:, :]
    for u in range(x_ref.shape[0]):
        out_ref[u] = jax.lax.dot_general(
            w, x_ref[u], (((1,), (0,)), ((), ())),
            preferred_element_type=jnp.float32)


def kernel(x, L, mapping1, idx_r, mapping2, sums, p_next_a, p_next_b):
    del sums, p_next_b
    B, n, Jr, T_, two = x.shape
    nt = T_ // 128
    x4 = x.reshape(n, Jr, nt, 128, two).transpose(0, 1, 2, 4, 3).reshape(
        n, Jr, nt * two, 128)
    m1 = mapping1.astype(jnp.int32).reshape(1, Jr)
    ir = idx_r.astype(jnp.int32).reshape(1, Jr)
    m2 = mapping2.astype(jnp.int32).reshape(_P, 1)
    pa = p_next_a.astype(jnp.int32).reshape(_P, 1)
    out4 = pl.pallas_call(
        _mix_kernel,
        grid=(n // 4,),
        in_specs=[
            pl.BlockSpec((4, Jr, nt * two, 128), lambda i: (i, 0, 0, 0)),
            pl.BlockSpec((_J, _J), lambda i: (0, 0)),
            pl.BlockSpec((1, Jr), lambda i: (0, 0)),
            pl.BlockSpec((1, Jr), lambda i: (0, 0)),
            pl.BlockSpec((_P, 1), lambda i: (0, 0)),
            pl.BlockSpec((_P, 1), lambda i: (0, 0)),
        ],
        out_specs=pl.BlockSpec((4, _P, nt * two, 128), lambda i: (i, 0, 0, 0)),
        out_shape=jax.ShapeDtypeStruct((n, _P, nt * two, 128), jnp.float32),
        scratch_shapes=[pltpu.VMEM((_P, _JR), jnp.float32)],
    )(x4, L, m1, ir, m2, pa)
    out = out4.reshape(n, _P, nt, two, 128).transpose(0, 1, 2, 4, 3).reshape(
        B, n, _P, T_, two)
    return out


# speedup vs baseline: 12.0577x; 1.3856x over previous
import jax
import jax.numpy as jnp
from jax.experimental import pallas as pl
from jax.experimental.pallas import tpu as pltpu

_J = 8
_M = 8
_JR = 28
_P = 56
_N = 8
_T = 2048


def _mix_kernel(x_ref, L_ref, m1_ref, ir_ref, m2_ref, pa_ref, out_ref, w_ref):
    @pl.when(pl.program_id(0) == 0)
    def _build_w():
        m1 = m1_ref[:].reshape(1, _JR)
        ir = ir_ref[:].reshape(1, _JR)
        m2 = m2_ref[:].reshape(_P, 1)
        pa = pa_ref[:].reshape(_P, 1)
        mask = (m2 == m1).astype(jnp.float32)
        oh_a = (pa == jax.lax.broadcasted_iota(jnp.int32, (_P, _J), 1)).astype(
            jnp.float32)
        oh_r = (jax.lax.broadcasted_iota(jnp.int32, (_J, _JR), 0) == ir).astype(
            jnp.float32)
        lg = jnp.dot(jnp.dot(oh_a, L_ref[:, :],
                             preferred_element_type=jnp.float32),
                     oh_r, preferred_element_type=jnp.float32)
        w_ref[:, :] = lg * mask

    w = w_ref[:, :]
    for u in range(x_ref.shape[0]):
        out_ref[u] = jax.lax.dot_general(
            w, x_ref[u], (((1,), (0,)), ((), ())),
            preferred_element_type=jnp.float32)


def kernel(x, L, mapping1, idx_r, mapping2, sums, p_next_a, p_next_b):
    del sums, p_next_b
    B, n, Jr, T_, two = x.shape
    nt = T_ // 128
    x4 = x.reshape(n, Jr, nt, 128, two).transpose(0, 1, 2, 4, 3).reshape(
        n, Jr, nt * two, 128)
    m1 = mapping1.astype(jnp.int32)
    ir = idx_r.astype(jnp.int32)
    m2 = mapping2.astype(jnp.int32)
    pa = p_next_a.astype(jnp.int32)
    out4 = pl.pallas_call(
        _mix_kernel,
        grid=(n // 4,),
        in_specs=[
            pl.BlockSpec((4, Jr, nt * two, 128), lambda i: (i, 0, 0, 0)),
            pl.BlockSpec((_J, _J), lambda i: (0, 0)),
            pl.BlockSpec((Jr,), lambda i: (0,)),
            pl.BlockSpec((Jr,), lambda i: (0,)),
            pl.BlockSpec((_P,), lambda i: (0,)),
            pl.BlockSpec((_P,), lambda i: (0,)),
        ],
        out_specs=pl.BlockSpec((4, _P, nt * two, 128), lambda i: (i, 0, 0, 0)),
        out_shape=jax.ShapeDtypeStruct((n, _P, nt * two, 128), jnp.float32),
        scratch_shapes=[pltpu.VMEM((_P, _JR), jnp.float32)],
    )(x4, L, m1, ir, m2, pa)
    out = out4.reshape(n, _P, nt, two, 128).transpose(0, 1, 2, 4, 3).reshape(
        B, n, _P, T_, two)
    return out
